# Initial kernel scaffold; baseline (speedup 1.0000x reference)
import jax
import jax.numpy as jnp
from jax.experimental import pallas as pl


def kernel(input_data, edge_index, connection_weights, biases):
    raise NotImplementedError("write your pallas kernel here")


# speedup vs baseline: 6.4764x; 6.4764x over previous
import functools

import jax
import jax.numpy as jnp
from jax import lax
from jax.experimental import pallas as pl
from jax.experimental.pallas import tpu as pltpu
from jax.experimental.pallas import tpu_sc as plsc

N = 2048
IN_SZ = 128
OUT_SZ = 64
OUT_BASE = N - OUT_SZ
L = 16
NC, NS = 2, 16
NW = NC * NS
ROWS_PER = N // NW


def _build_weight_mats(edge_index, weights):
    e = weights.shape[0]
    chunks = e // L
    mesh = plsc.VectorSubcoreMesh(
        core_axis_name="c", subcore_axis_name="s", num_cores=NC, num_subcores=NS
    )

    @functools.partial(
        pl.kernel,
        out_type=[
            jax.ShapeDtypeStruct((N * IN_SZ,), jnp.float32),
            jax.ShapeDtypeStruct((N * OUT_SZ,), jnp.float32),
        ],
        mesh=mesh,
        compiler_params=pltpu.CompilerParams(needs_layout_passes=False),
        scratch_types=[
            pltpu.VMEM((e,), jnp.int32),
            pltpu.VMEM((e,), jnp.int32),
            pltpu.VMEM((e,), jnp.float32),
            pltpu.VMEM((ROWS_PER * IN_SZ,), jnp.float32),
            pltpu.VMEM((ROWS_PER * OUT_SZ,), jnp.float32),
        ],
    )
    def build(edge_hbm, w_hbm, w1t_hbm, w2_hbm, src_v, dst_v, w_v, w1_v, w2_v):
        wid = lax.axis_index("s") * NC + lax.axis_index("c")
        lo = wid * ROWS_PER

        pltpu.sync_copy(edge_hbm.at[0], src_v)
        pltpu.sync_copy(edge_hbm.at[1], dst_v)
        pltpu.sync_copy(w_hbm, w_v)

        zf = jnp.zeros((L,), jnp.float32)

        def zero1(i, _):
            w1_v[pl.ds(i * L, L)] = zf
            return 0

        lax.fori_loop(0, ROWS_PER * IN_SZ // L, zero1, 0)

        def zero2(i, _):
            w2_v[pl.ds(i * L, L)] = zf
            return 0

        lax.fori_loop(0, ROWS_PER * OUT_SZ // L, zero2, 0)

        def body(i, _):
            base = i * L
            s = src_v[pl.ds(base, L)]
            d = dst_v[pl.ds(base, L)]
            w = w_v[pl.ds(base, L)]
            d_rel = d - lo
            m1 = (s < IN_SZ) & (d_rel >= 0) & (d_rel < ROWS_PER)
            idx1 = jnp.where(m1, d_rel * IN_SZ + s, 0)
            plsc.addupdate_scatter(w1_v, [idx1], w, mask=m1)
            s_rel = s - lo
            j = d - OUT_BASE
            m2 = (j >= 0) & (s_rel >= 0) & (s_rel < ROWS_PER)
            idx2 = jnp.where(m2, s_rel * OUT_SZ + j, 0)
            plsc.addupdate_scatter(w2_v, [idx2], w, mask=m2)
            return 0

        lax.fori_loop(0, chunks, body, 0)

        pltpu.sync_copy(w1_v, w1t_hbm.at[pl.ds(lo * IN_SZ, ROWS_PER * IN_SZ)])
        pltpu.sync_copy(w2_v, w2_hbm.at[pl.ds(lo * OUT_SZ, ROWS_PER * OUT_SZ)])

    return build(edge_index, weights)


def _dense_steps(inp_ref, w1t_ref, w2_ref, b_ref, b2_ref, out_ref):
    act1 = jnp.tanh(
        lax.dot_general(
            inp_ref[...],
            w1t_ref[...],
            (((1,), (1,)), ((), ())),
            preferred_element_type=jnp.float32,
            precision=lax.Precision.HIGHEST,
        )
        + b_ref[...]
    )
    out_ref[...] = jnp.tanh(
        lax.dot_general(
            act1,
            w2_ref[...],
            (((1,), (0,)), ((), ())),
            preferred_element_type=jnp.float32,
            precision=lax.Precision.HIGHEST,
        )
        + b2_ref[...]
    )


def kernel(input_data, edge_index, connection_weights, biases):
    b = input_data.shape[0]
    bp = max(8, -(-b // 8) * 8)
    w1t_flat, w2_flat = _build_weight_mats(edge_index, connection_weights)
    w1t = w1t_flat.reshape(N, IN_SZ)
    w2 = w2_flat.reshape(N, OUT_SZ)
    inp = jnp.zeros((bp, IN_SZ), jnp.float32).at[:b].set(input_data)
    bias_row = biases.reshape(1, N)
    bias_out = biases[-OUT_SZ:].reshape(1, OUT_SZ)
    out = pl.pallas_call(
        _dense_steps,
        out_shape=jax.ShapeDtypeStruct((bp, OUT_SZ), jnp.float32),
    )(inp, w1t, w2, bias_row, bias_out)
    return out[:b]


# speedup vs baseline: 9.7299x; 1.5024x over previous
import functools

import jax
import jax.numpy as jnp
from jax import lax
from jax.experimental import pallas as pl
from jax.experimental.pallas import tpu as pltpu
from jax.experimental.pallas import tpu_sc as plsc

N = 2048
IN_SZ = 128
OUT_SZ = 64
OUT_BASE = N - OUT_SZ
L = 16
NS = 16


def _tanh16(x):
    ex = jnp.exp(-2.0 * jnp.abs(x))
    return jnp.sign(x) * (1.0 - ex) / (1.0 + ex)


def _brain_sc(src, dst, weights, inp_flat, biases, zeros, batch):
    e = weights.shape[0]
    e_per = e // NS
    chunks = e_per // L
    act_sz = batch * N
    out_sz = batch * OUT_SZ
    red_per = act_sz // NS
    outred_per = out_sz // NS
    mesh = plsc.VectorSubcoreMesh(
        core_axis_name="c", subcore_axis_name="s", num_cores=1, num_subcores=NS
    )

    @functools.partial(
        pl.kernel,
        out_type=jax.ShapeDtypeStruct((out_sz,), jnp.float32),
        mesh=mesh,
        compiler_params=pltpu.CompilerParams(needs_layout_passes=False),
        scratch_types=[
            pltpu.VMEM((e_per,), jnp.int32),
            pltpu.VMEM((e_per,), jnp.int32),
            pltpu.VMEM((e_per,), jnp.float32),
            pltpu.VMEM((batch * IN_SZ,), jnp.float32),
            pltpu.VMEM((act_sz,), jnp.float32),
            pltpu.VMEM((act_sz,), jnp.float32),
            pltpu.VMEM((NS * red_per,), jnp.float32),
            pltpu.VMEM((red_per,), jnp.float32),
            pltpu.VMEM((out_sz,), jnp.float32),
            pltpu.VMEM((NS * out_sz,), jnp.float32),
            pltpu.VMEM((L,), jnp.float32),
            pltpu.VMEM_SHARED((NS * act_sz,), jnp.float32),
            pltpu.VMEM_SHARED((act_sz,), jnp.float32),
            pltpu.VMEM_SHARED((NS * out_sz,), jnp.float32),
        ],
    )
    def run(src_hbm, dst_hbm, w_hbm, in_hbm, b_hbm, z_hbm, out_hbm,
            src_v, dst_v, w_v, in_v, part_v, act1_v, red_v, slice_v,
            outp_v, outred_v, fin_v, parts_sh, act1_sh, outparts_sh):
        tid = lax.axis_index("s")
        ebase = tid * e_per

        pltpu.sync_copy(src_hbm.at[pl.ds(ebase, e_per)], src_v)
        pltpu.sync_copy(dst_hbm.at[pl.ds(ebase, e_per)], dst_v)
        pltpu.sync_copy(w_hbm.at[pl.ds(ebase, e_per)], w_v)
        pltpu.sync_copy(in_hbm, in_v)
        pltpu.sync_copy(z_hbm.at[pl.ds(0, act_sz)], part_v)
        pltpu.sync_copy(z_hbm.at[pl.ds(0, out_sz)], outp_v)

        def step1(i, _):
            s = src_v[pl.ds(i * L, L)]
            d = dst_v[pl.ds(i * L, L)]
            w = w_v[pl.ds(i * L, L)]
            m = s < IN_SZ
            s_c = jnp.where(m, s, 0)
            for bb in range(batch):
                val = plsc.load_gather(in_v, [s_c + (bb * IN_SZ)]) * w
                plsc.addupdate_scatter(part_v, [d + (bb * N)], val, mask=m)
            return 0

        lax.fori_loop(0, chunks, step1, 0)

        pltpu.sync_copy(part_v, parts_sh.at[pl.ds(tid * act_sz, act_sz)])
        plsc.subcore_barrier()
        rbase = tid * red_per
        for p in range(NS):
            pltpu.sync_copy(
                parts_sh.at[pl.ds(p * act_sz + rbase, red_per)],
                red_v.at[pl.ds(p * red_per, red_per)],
            )
        pltpu.sync_copy(b_hbm.at[pl.ds(rbase % N, red_per)], slice_v)

        def reduce1(i, _):
            acc = red_v[pl.ds(i * L, L)]
            for p in range(1, NS):
                acc = acc + red_v[pl.ds(p * red_per + i * L, L)]
            slice_v[pl.ds(i * L, L)] = _tanh16(acc + slice_v[pl.ds(i * L, L)])
            return 0

        lax.fori_loop(0, red_per // L, reduce1, 0)

        pltpu.sync_copy(slice_v, act1_sh.at[pl.ds(rbase, red_per)])
        plsc.subcore_barrier()
        pltpu.sync_copy(act1_sh, act1_v)

        def step2(i, _):
            s = src_v[pl.ds(i * L, L)]
            d = dst_v[pl.ds(i * L, L)]
            w = w_v[pl.ds(i * L, L)]
            m = d >= OUT_BASE
            j = jnp.where(m, d - OUT_BASE, 0)
            for bb in range(batch):
                val = plsc.load_gather(act1_v, [s + (bb * N)]) * w
                plsc.addupdate_scatter(outp_v, [j + (bb * OUT_SZ)], val, mask=m)
            return 0

        lax.fori_loop(0, chunks, step2, 0)

        pltpu.sync_copy(outp_v, outparts_sh.at[pl.ds(tid * out_sz, out_sz)])
        plsc.subcore_barrier()
        pltpu.sync_copy(outparts_sh, outred_v)
        obase = tid * outred_per

        pltpu.sync_copy(b_hbm.at[pl.ds(OUT_BASE, OUT_SZ)], slice_v.at[pl.ds(0, OUT_SZ)])

        def reduce_out(i, _):
            f = obase + i * L
            acc = outred_v[pl.ds(f, L)]
            for p in range(1, NS):
                acc = acc + outred_v[pl.ds(p * out_sz + f, L)]
            bv = slice_v[pl.ds(f % OUT_SZ, L)]
            fin_v[...] = _tanh16(acc + bv)
            pltpu.sync_copy(fin_v, out_hbm.at[pl.ds(f, L)])
            return 0

        lax.fori_loop(0, outred_per // L, reduce_out, 0)

    return run(src, dst, weights, inp_flat, biases, zeros)


def kernel(input_data, edge_index, connection_weights, biases):
    b = input_data.shape[0]
    zeros = jnp.zeros((b * N,), jnp.float32)
    out_flat = _brain_sc(
        edge_index[0],
        edge_index[1],
        connection_weights,
        input_data.reshape(-1),
        biases,
        zeros,
        b,
    )
    return out_flat.reshape(b, OUT_SZ)


# speedup vs baseline: 9.8313x; 1.0104x over previous
import functools

import jax
import jax.numpy as jnp
from jax import lax
from jax.experimental import pallas as pl
from jax.experimental.pallas import tpu as pltpu
from jax.experimental.pallas import tpu_sc as plsc

N = 2048
IN_SZ = 128
OUT_SZ = 64
OUT_BASE = N - OUT_SZ
L = 16
NS = 16
UN = 2


def _tanh16(x):
    ex = jnp.exp(-2.0 * jnp.abs(x))
    return jnp.sign(x) * (1.0 - ex) / (1.0 + ex)


def _brain_sc(edge_index, weights, input_data, biases, zeros):
    e = weights.shape[0]
    batch = input_data.shape[0]
    e_per = e // NS
    chunks = e_per // L
    act_sz = batch * N
    out_sz = batch * OUT_SZ
    red_per = act_sz // NS
    outred_per = out_sz // NS
    mesh = plsc.VectorSubcoreMesh(
        core_axis_name="c", subcore_axis_name="s", num_cores=1, num_subcores=NS
    )

    @functools.partial(
        pl.kernel,
        out_type=jax.ShapeDtypeStruct((out_sz,), jnp.float32),
        mesh=mesh,
        compiler_params=pltpu.CompilerParams(needs_layout_passes=False),
        scratch_types=[
            pltpu.VMEM((e_per,), jnp.int32),
            pltpu.VMEM((e_per,), jnp.int32),
            pltpu.VMEM((e_per,), jnp.float32),
            pltpu.VMEM((batch * IN_SZ,), jnp.float32),
            pltpu.VMEM((act_sz,), jnp.float32),
            pltpu.VMEM((act_sz,), jnp.float32),
            pltpu.VMEM((NS, red_per), jnp.float32),
            pltpu.VMEM((red_per,), jnp.float32),
            pltpu.VMEM((out_sz,), jnp.float32),
            pltpu.VMEM((NS * out_sz,), jnp.float32),
            pltpu.VMEM((L,), jnp.float32),
            pltpu.VMEM_SHARED((NS, act_sz), jnp.float32),
            pltpu.VMEM_SHARED((act_sz,), jnp.float32),
            pltpu.VMEM_SHARED((NS * out_sz,), jnp.float32),
        ],
    )
    def run(edge_hbm, w_hbm, in_hbm, b_hbm, z_hbm, out_hbm,
            src_v, dst_v, w_v, in_v, part_v, act1_v, red_v, slice_v,
            outp_v, outred_v, fin_v, parts_sh, act1_sh, outparts_sh):
        tid = lax.axis_index("s")
        ebase = tid * e_per

        pltpu.sync_copy(edge_hbm.at[0, pl.ds(ebase, e_per)], src_v)
        pltpu.sync_copy(edge_hbm.at[1, pl.ds(ebase, e_per)], dst_v)
        pltpu.sync_copy(w_hbm.at[pl.ds(ebase, e_per)], w_v)
        for bb in range(batch):
            pltpu.sync_copy(in_hbm.at[bb], in_v.at[pl.ds(bb * IN_SZ, IN_SZ)])
        pltpu.sync_copy(z_hbm.at[pl.ds(0, act_sz)], part_v)
        pltpu.sync_copy(z_hbm.at[pl.ds(0, out_sz)], outp_v)

        def one_chunk1(i):
            s = src_v[pl.ds(i * L, L)]
            d = dst_v[pl.ds(i * L, L)]
            w = w_v[pl.ds(i * L, L)]
            m = s < IN_SZ
            s_c = jnp.where(m, s, 0)
            for bb in range(batch):
                val = plsc.load_gather(in_v, [s_c + (bb * IN_SZ)]) * w
                plsc.addupdate_scatter(part_v, [d + (bb * N)], val, mask=m)

        def step1(i, _):
            for u in range(UN):
                one_chunk1(i * UN + u)
            return 0

        lax.fori_loop(0, chunks // UN, step1, 0)

        pltpu.sync_copy(part_v, parts_sh.at[tid])
        plsc.subcore_barrier()
        rbase = tid * red_per
        pltpu.sync_copy(parts_sh.at[:, pl.ds(rbase, red_per)], red_v)
        pltpu.sync_copy(b_hbm.at[pl.ds(rbase % N, red_per)], slice_v)

        def reduce1(i, _):
            acc = red_v[0, pl.ds(i * L, L)]
            for p in range(1, NS):
                acc = acc + red_v[p, pl.ds(i * L, L)]
            slice_v[pl.ds(i * L, L)] = _tanh16(acc + slice_v[pl.ds(i * L, L)])
            return 0

        lax.fori_loop(0, red_per // L, reduce1, 0)

        pltpu.sync_copy(slice_v, act1_sh.at[pl.ds(rbase, red_per)])
        plsc.subcore_barrier()
        pltpu.sync_copy(act1_sh, act1_v)

        def one_chunk2(i):
            s = src_v[pl.ds(i * L, L)]
            d = dst_v[pl.ds(i * L, L)]
            w = w_v[pl.ds(i * L, L)]
            m = d >= OUT_BASE
            j = jnp.where(m, d - OUT_BASE, 0)
            for bb in range(batch):
                val = plsc.load_gather(act1_v, [s + (bb * N)]) * w
                plsc.addupdate_scatter(outp_v, [j + (bb * OUT_SZ)], val, mask=m)

        def step2(i, _):
            for u in range(UN):
                one_chunk2(i * UN + u)
            return 0

        lax.fori_loop(0, chunks // UN, step2, 0)

        pltpu.sync_copy(outp_v, outparts_sh.at[pl.ds(tid * out_sz, out_sz)])
        plsc.subcore_barrier()
        obase = tid * outred_per
        pltpu.sync_copy(outparts_sh, outred_v)
        pltpu.sync_copy(
            b_hbm.at[pl.ds(OUT_BASE + (obase % OUT_SZ), L)],
            fin_v,
        )
        acc = fin_v[...]
        for p in range(NS):
            acc = acc + outred_v[pl.ds(p * out_sz + obase, L)]
        fin_v[...] = _tanh16(acc)
        pltpu.sync_copy(fin_v, out_hbm.at[pl.ds(obase, L)])

    return run(edge_index, weights, input_data, biases, zeros)


def kernel(input_data, edge_index, connection_weights, biases):
    b = input_data.shape[0]
    zeros = jnp.zeros((b * N,), jnp.float32)
    out = _brain_sc(edge_index, connection_weights, input_data, biases, zeros)
    return out.reshape(b, OUT_SZ)


# speedup vs baseline: 11.6625x; 1.1863x over previous
import functools

import jax
import jax.numpy as jnp
from jax import lax
from jax.experimental import pallas as pl
from jax.experimental.pallas import tpu as pltpu
from jax.experimental.pallas import tpu_sc as plsc

N = 2048
IN_SZ = 128
OUT_SZ = 64
OUT_BASE = N - OUT_SZ
L = 16
NS = 16
UN = 2


def _tanh16(x):
    ex = jnp.exp(-2.0 * jnp.abs(x))
    return jnp.sign(x) * (1.0 - ex) / (1.0 + ex)


def _brain_sc(edge_index, weights, input_data, biases, zeros):
    e = weights.shape[0]
    batch = input_data.shape[0]
    e_per = e // NS
    chunks = e_per // L
    act_sz = batch * N
    out_sz = batch * OUT_SZ
    red_per = act_sz // NS
    outred_per = out_sz // NS
    mesh = plsc.VectorSubcoreMesh(
        core_axis_name="c", subcore_axis_name="s", num_cores=1, num_subcores=NS
    )

    @functools.partial(
        pl.kernel,
        out_type=jax.ShapeDtypeStruct((batch, OUT_SZ), jnp.float32),
        mesh=mesh,
        compiler_params=pltpu.CompilerParams(needs_layout_passes=False),
        scratch_types=[
            pltpu.VMEM((e_per,), jnp.int32),
            pltpu.VMEM((e_per,), jnp.int32),
            pltpu.VMEM((e_per,), jnp.float32),
            pltpu.VMEM((batch, IN_SZ), jnp.float32),
            pltpu.VMEM((act_sz,), jnp.float32),
            pltpu.VMEM((act_sz,), jnp.float32),
            pltpu.VMEM((NS, red_per), jnp.float32),
            pltpu.VMEM((red_per,), jnp.float32),
            pltpu.VMEM((out_sz,), jnp.float32),
            pltpu.VMEM((NS * out_sz,), jnp.float32),
            pltpu.VMEM((OUT_SZ,), jnp.float32),
            pltpu.VMEM_SHARED((NS, act_sz), jnp.float32),
            pltpu.VMEM_SHARED((act_sz,), jnp.float32),
            pltpu.VMEM_SHARED((NS * out_sz,), jnp.float32),
            pltpu.SemaphoreType.DMA,
        ],
    )
    def run(edge_hbm, w_hbm, in_hbm, b_hbm, z_hbm, out_hbm,
            src_v, dst_v, w_v, in_v, part_v, act1_v, red_v, slice_v,
            outp_v, outred_v, fin_v, parts_sh, act1_sh, outparts_sh, sem):
        tid = lax.axis_index("s")
        ebase = tid * e_per
        rbase = tid * red_per

        copies = [
            pltpu.async_copy(edge_hbm.at[0, pl.ds(ebase, e_per)], src_v, sem),
            pltpu.async_copy(edge_hbm.at[1, pl.ds(ebase, e_per)], dst_v, sem),
            pltpu.async_copy(w_hbm.at[pl.ds(ebase, e_per)], w_v, sem),
            pltpu.async_copy(in_hbm, in_v, sem),
            pltpu.async_copy(z_hbm.at[pl.ds(0, act_sz)], part_v, sem),
            pltpu.async_copy(z_hbm.at[pl.ds(0, out_sz)], outp_v, sem),
            pltpu.async_copy(b_hbm.at[pl.ds(rbase % N, red_per)], slice_v, sem),
        ]
        for c in copies:
            c.wait()

        def one_chunk1(i):
            s = src_v[pl.ds(i * L, L)]
            d = dst_v[pl.ds(i * L, L)]
            w = w_v[pl.ds(i * L, L)]
            m = s < IN_SZ
            s_c = jnp.where(m, s, 0)
            for bb in range(batch):
                row = jnp.full((L,), bb, jnp.int32)
                val = plsc.load_gather(in_v, [row, s_c]) * w
                plsc.addupdate_scatter(part_v, [d + (bb * N)], val, mask=m)

        def step1(i, _):
            for u in range(UN):
                one_chunk1(i * UN + u)
            return 0

        lax.fori_loop(0, chunks // UN, step1, 0)

        pltpu.sync_copy(part_v, parts_sh.at[tid])
        plsc.subcore_barrier()
        pltpu.sync_copy(parts_sh.at[:, pl.ds(rbase, red_per)], red_v)

        def reduce1(i, _):
            acc = red_v[0, pl.ds(i * L, L)]
            for p in range(1, NS):
                acc = acc + red_v[p, pl.ds(i * L, L)]
            slice_v[pl.ds(i * L, L)] = _tanh16(acc + slice_v[pl.ds(i * L, L)])
            return 0

        lax.fori_loop(0, red_per // L, reduce1, 0)

        pltpu.sync_copy(slice_v, act1_sh.at[pl.ds(rbase, red_per)])
        plsc.subcore_barrier()
        pltpu.sync_copy(act1_sh, act1_v)

        def one_chunk2(i):
            s = src_v[pl.ds(i * L, L)]
            d = dst_v[pl.ds(i * L, L)]
            w = w_v[pl.ds(i * L, L)]
            m = d >= OUT_BASE
            j = jnp.where(m, d - OUT_BASE, 0)
            for bb in range(batch):
                val = plsc.load_gather(act1_v, [s + (bb * N)]) * w
                plsc.addupdate_scatter(outp_v, [j + (bb * OUT_SZ)], val, mask=m)

        def step2(i, _):
            for u in range(UN):
                one_chunk2(i * UN + u)
            return 0

        lax.fori_loop(0, chunks // UN, step2, 0)

        pltpu.sync_copy(outp_v, outparts_sh.at[pl.ds(tid * out_sz, out_sz)])
        plsc.subcore_barrier()

        @pl.when(tid < batch)
        def _():
            pltpu.sync_copy(outparts_sh, outred_v)
            pltpu.sync_copy(b_hbm.at[pl.ds(OUT_BASE, OUT_SZ)],
                            fin_v)
            obase = tid * OUT_SZ
            for i in range(OUT_SZ // L):
                acc = fin_v[pl.ds(i * L, L)]
                for p in range(NS):
                    acc = acc + outred_v[pl.ds(p * out_sz + obase + i * L, L)]
                fin_v[pl.ds(i * L, L)] = _tanh16(acc)
            pltpu.sync_copy(fin_v, out_hbm.at[tid])

    return run(edge_index, weights, input_data, biases, zeros)


def kernel(input_data, edge_index, connection_weights, biases):
    b = input_data.shape[0]
    zeros = jnp.zeros((b * N,), jnp.float32)
    return _brain_sc(edge_index, connection_weights, input_data, biases, zeros)
